# baseline (speedup 1.0000x reference)
import functools

import jax
import jax.numpy as jnp
from jax import lax
from jax.experimental import pallas as pl
from jax.experimental.pallas import tpu as pltpu
from jax.experimental.pallas import tpu_sc as plsc

B = 16384
F = 26
V = 100000
D = 16

NC = 2
NS = 16
NW = NC * NS
BPW = B // NW
IPW = BPW * F
NROW = IPW // 128


def _sc_lin_body(idx_hbm, wflat_hbm, out_hbm, idx_v, vals_v, lin_v, sem):
    wid = lax.axis_index("s") * NC + lax.axis_index("c")
    pltpu.sync_copy(idx_hbm.at[wid], idx_v)
    def _gather_chunk(step, carry):
        descs = [
            pltpu.async_copy(
                wflat_hbm.at[idx_v.at[step * 8 + t]], vals_v.at[step * 8 + t], sem
            )
            for t in range(8)
        ]
        for dsc in descs:
            dsc.wait()
        return carry

    lax.fori_loop(0, NROW // 8, _gather_chunk, 0)
    for c in range(BPW // 16):
        r0 = c // 8
        col = (c % 8) * 16
        acc = vals_v[r0, pl.ds(col, 16)]
        for f in range(1, F):
            acc = acc + vals_v[f * 4 + r0, pl.ds(col, 16)]
        lin_v[pl.ds(c * 16, 16)] = acc
    pltpu.sync_copy(lin_v, out_hbm.at[pl.ds(wid * BPW, BPW)])


@functools.partial(jax.jit, static_argnums=())
def _sc_linear(idx, wflat):
    mesh = plsc.VectorSubcoreMesh(core_axis_name="c", subcore_axis_name="s")
    return pl.kernel(
        _sc_lin_body,
        out_type=jax.ShapeDtypeStruct((B,), jnp.float32),
        mesh=mesh,
        scratch_types=[
            pltpu.VMEM((NROW, 128), jnp.int32),
            pltpu.VMEM((NROW, 128), jnp.float32),
            pltpu.VMEM((BPW,), jnp.float32),
            pltpu.SemaphoreType.DMA,
        ],
    )(idx, wflat)


def _tc_inter_body(fe_ref, sel_ref, out_ref):
    blk = fe_ref[...]
    sel = sel_ref[...]
    s = jnp.dot(blk, sel, preferred_element_type=jnp.float32)
    q = jnp.dot(blk * blk, sel, preferred_element_type=jnp.float32)
    out_ref[...] = (s * s - q) * 0.5


def _tc_interaction(fe2d, sel):
    bt = 2048
    return pl.pallas_call(
        _tc_inter_body,
        grid=(B // bt,),
        in_specs=[
            pl.BlockSpec((bt, F * D), lambda i: (i, 0)),
            pl.BlockSpec((F * D, D), lambda i: (0, 0)),
        ],
        out_specs=pl.BlockSpec((bt, D), lambda i: (i, 0)),
        out_shape=jax.ShapeDtypeStruct((B, D), jnp.float32),
    )(fe2d, sel)


def kernel(x, feature_emb, w_linear, bias):
    fe2d = feature_emb.reshape(B, F * D)
    sel = jnp.tile(jnp.eye(D, dtype=jnp.float32), (F, 1))
    wflat = w_linear.reshape(F * V)
    fidx = x.T.astype(jnp.int32) + (jnp.arange(F, dtype=jnp.int32) * V)[:, None]
    fidx = fidx.reshape(F, NW, BPW).transpose(1, 0, 2).reshape(NW, NROW, 128)
    lin = _sc_linear(fidx, wflat)
    inter = _tc_interaction(fe2d, sel)
    return inter + (lin + bias[0])[:, None]


# speedup vs baseline: 1.4628x; 1.4628x over previous
import jax
import jax.numpy as jnp
from jax import lax
from jax.experimental import pallas as pl
from jax.experimental.pallas import tpu as pltpu
from jax.experimental.pallas import tpu_sc as plsc

B = 16384
F = 26
V = 100000
D = 16

NC = 2
NS = 16
NW = NC * NS
BPW = B // NW
IPW = BPW * F
NROW = IPW // 128
K = 8


def _sc_lin_body(idx_hbm, wflat_hbm, out_hbm, idx_v, vals_v, lin_v, sem):
    wid = lax.axis_index("s") * NC + lax.axis_index("c")
    pltpu.sync_copy(idx_hbm.at[wid], idx_v)

    for t in range(K):
        pltpu.async_copy(wflat_hbm.at[idx_v.at[t]], vals_v.at[t], sem)

    def chunk(i, carry):
        base = i * K
        for t in range(K):
            j = base + K + t
            pltpu.async_copy(wflat_hbm.at[idx_v.at[j]], vals_v.at[j], sem)
        for t in range(K):
            j = base + t
            pltpu.make_async_copy(
                wflat_hbm.at[idx_v.at[j]], vals_v.at[j], sem
            ).wait()
        return carry

    lax.fori_loop(0, NROW // K - 1, chunk, 0)
    for t in range(K):
        j = NROW - K + t
        pltpu.make_async_copy(wflat_hbm.at[idx_v.at[j]], vals_v.at[j], sem).wait()

    for c in range(BPW // 16):
        r0 = c // 8
        col = (c % 8) * 16
        acc = vals_v[r0, pl.ds(col, 16)]
        for f in range(1, F):
            acc = acc + vals_v[f * 4 + r0, pl.ds(col, 16)]
        lin_v[pl.ds(c * 16, 16)] = acc
    pltpu.sync_copy(lin_v, out_hbm.at[pl.ds(wid * BPW, BPW)])


def _sc_linear(idx, wflat):
    mesh = plsc.VectorSubcoreMesh(core_axis_name="c", subcore_axis_name="s")
    return pl.kernel(
        _sc_lin_body,
        out_type=jax.ShapeDtypeStruct((B,), jnp.float32),
        mesh=mesh,
        scratch_types=[
            pltpu.VMEM((NROW, 128), jnp.int32),
            pltpu.VMEM((NROW, 128), jnp.float32),
            pltpu.VMEM((BPW,), jnp.float32),
            pltpu.SemaphoreType.DMA,
        ],
    )(idx, wflat)


def _tc_inter_body(fe_ref, out_ref):
    acc = fe_ref[0]
    acc2 = acc * acc
    for f in range(1, F):
        v = fe_ref[f]
        acc = acc + v
        acc2 = acc2 + v * v
    out_ref[...] = (acc * acc - acc2) * 0.5


def _tc_interaction(fe_t):
    bt = 2048
    return pl.pallas_call(
        _tc_inter_body,
        grid=(B // bt,),
        in_specs=[pl.BlockSpec((F, D, bt), lambda i: (0, 0, i))],
        out_specs=pl.BlockSpec((D, bt), lambda i: (0, i)),
        out_shape=jax.ShapeDtypeStruct((D, B), jnp.float32),
    )(fe_t)


def kernel(x, feature_emb, w_linear, bias):
    fe_t = feature_emb.transpose(1, 2, 0)
    wflat = w_linear.reshape(F * V)
    fidx = x.T.astype(jnp.int32) + (jnp.arange(F, dtype=jnp.int32) * V)[:, None]
    fidx = fidx.reshape(F, NW, BPW).transpose(1, 0, 2).reshape(NW, NROW, 128)
    lin = _sc_linear(fidx, wflat)
    inter_t = _tc_interaction(fe_t)
    return (inter_t + (lin + bias[0])[None, :]).T
